# initial kernel scaffold (unmeasured)
import functools

import jax
import jax.numpy as jnp
from jax import lax
from jax.experimental import pallas as pl
from jax.experimental.pallas import tpu as pltpu

N_DEV = 16


def kernel(x, w_mat, scale_x, scale_w):
    m_total, k_per = x.shape
    k_total, n = w_mat.shape
    m_per = m_total // N_DEV

    def body(x_ref, w_ref, sx_ref, sw_ref, out_ref,
             x8_ref, xfull_ref, send_sems, recv_sems):
        my = lax.axis_index("i")

        x8_ref[...] = x_ref[...].astype(jnp.float8_e5m2)

        barrier = pltpu.get_barrier_semaphore()
        for d in range(1, N_DEV):
            peer = lax.rem(my + d, N_DEV)
            pl.semaphore_signal(barrier, inc=1, device_id=(peer,),
                                device_id_type=pl.DeviceIdType.MESH)
        pl.semaphore_wait(barrier, N_DEV - 1)

        xfull_ref[:, pl.ds(my * k_per, k_per)] = x8_ref[pl.ds(my * m_per, m_per), :]

        sends = []
        for d in range(1, N_DEV):
            peer = lax.rem(my + d, N_DEV)
            rdma = pltpu.make_async_remote_copy(
                src_ref=x8_ref.at[pl.ds(peer * m_per, m_per), :],
                dst_ref=xfull_ref.at[:, pl.ds(my * k_per, k_per)],
                send_sem=send_sems.at[d],
                recv_sem=recv_sems.at[my],
                device_id=(peer,),
                device_id_type=pl.DeviceIdType.MESH,
            )
            rdma.start()
            sends.append(rdma)

        for d in range(1, N_DEV):
            peer = lax.rem(my + d, N_DEV)
            recv = pltpu.make_async_remote_copy(
                src_ref=x8_ref.at[pl.ds(0, m_per), :],
                dst_ref=xfull_ref.at[:, pl.ds(peer * k_per, k_per)],
                send_sem=send_sems.at[d],
                recv_sem=recv_sems.at[peer],
                device_id=(peer,),
                device_id_type=pl.DeviceIdType.MESH,
            )
            recv.wait_recv()
        for rdma in sends:
            rdma.wait_send()

        acc = lax.dot_general(
            xfull_ref[...].astype(jnp.float32), w_ref[...],
            dimension_numbers=(((1,), (0,)), ((), ())),
            preferred_element_type=jnp.float32,
        )
        scale = sx_ref[0] * sw_ref[0]
        out_ref[...] = jnp.maximum(acc * scale, 0.0)

        @functools.partial(pl.run_scoped, sem=pltpu.SemaphoreType.REGULAR)
        def _(sem):
            for d in range(1, N_DEV):
                peer = lax.rem(my + d, N_DEV)
                pl.semaphore_signal(sem, inc=1, device_id=(peer,),
                                    device_id_type=pl.DeviceIdType.MESH)
            pl.semaphore_wait(sem, N_DEV - 1)

    return pl.pallas_call(
        body,
        out_shape=jax.ShapeDtypeStruct((m_per, n), jnp.float32),
        in_specs=[
            pl.BlockSpec(memory_space=pltpu.VMEM),
            pl.BlockSpec(memory_space=pltpu.VMEM),
            pl.BlockSpec(memory_space=pltpu.SMEM),
            pl.BlockSpec(memory_space=pltpu.SMEM),
        ],
        out_specs=pl.BlockSpec(memory_space=pltpu.VMEM),
        scratch_shapes=[
            pltpu.VMEM((m_total, k_per), jnp.float8_e5m2),
            pltpu.VMEM((m_per, k_total), jnp.float8_e5m2),
            pltpu.SemaphoreType.DMA((N_DEV,)),
            pltpu.SemaphoreType.DMA((N_DEV,)),
        ],
        compiler_params=pltpu.CompilerParams(collective_id=0),
    )(x, w_mat, scale_x, scale_w)


# baseline (device time: 44652 ns/iter reference)
import functools

import jax
import jax.numpy as jnp
from jax import lax
from jax.experimental import pallas as pl
from jax.experimental.pallas import tpu as pltpu

N_DEV = 16


def kernel(x, w_mat, scale_x, scale_w):
    m_total, k_per = x.shape
    k_total, n = w_mat.shape
    m_per = m_total // N_DEV

    def body(x_ref, w_ref, sx_ref, sw_ref, out_ref,
             x8_ref, xfull_ref, send_sems, recv_sems):
        my = lax.axis_index("i")

        x8_ref[...] = x_ref[...].astype(jnp.float8_e5m2)

        barrier = pltpu.get_barrier_semaphore()
        for d in range(1, N_DEV):
            peer = lax.rem(my + d, N_DEV)
            pl.semaphore_signal(barrier, inc=1, device_id=(peer,),
                                device_id_type=pl.DeviceIdType.MESH)
        pl.semaphore_wait(barrier, N_DEV - 1)

        xfull_ref[:, pl.ds(my * k_per, k_per)] = x8_ref[pl.ds(my * m_per, m_per), :]

        sends = []
        for d in range(1, N_DEV):
            peer = lax.rem(my + d, N_DEV)
            rdma = pltpu.make_async_remote_copy(
                src_ref=x8_ref.at[pl.ds(peer * m_per, m_per), :],
                dst_ref=xfull_ref.at[:, pl.ds(my * k_per, k_per)],
                send_sem=send_sems.at[d],
                recv_sem=recv_sems.at[my],
                device_id=(peer,),
                device_id_type=pl.DeviceIdType.MESH,
            )
            rdma.start()
            sends.append(rdma)

        for d in range(1, N_DEV):
            peer = lax.rem(my + d, N_DEV)
            recv = pltpu.make_async_remote_copy(
                src_ref=x8_ref.at[pl.ds(0, m_per), :],
                dst_ref=xfull_ref.at[:, pl.ds(peer * k_per, k_per)],
                send_sem=send_sems.at[d],
                recv_sem=recv_sems.at[peer],
                device_id=(peer,),
                device_id_type=pl.DeviceIdType.MESH,
            )
            recv.wait_recv()
        for rdma in sends:
            rdma.wait_send()

        acc = lax.dot_general(
            xfull_ref[...].astype(jnp.float32), w_ref[...],
            dimension_numbers=(((1,), (0,)), ((), ())),
            preferred_element_type=jnp.float32,
        )
        scale = sx_ref[0] * sw_ref[0]
        out_ref[...] = jnp.maximum(acc * scale, 0.0)

        @functools.partial(pl.run_scoped, sem=pltpu.SemaphoreType.REGULAR)
        def _(sem):
            for d in range(1, N_DEV):
                peer = lax.rem(my + d, N_DEV)
                pl.semaphore_signal(sem, inc=1, device_id=(peer,),
                                    device_id_type=pl.DeviceIdType.MESH)
            pl.semaphore_wait(sem, N_DEV - 1)

    return pl.pallas_call(
        body,
        out_shape=jax.ShapeDtypeStruct((m_per, n), jnp.float32),
        in_specs=[
            pl.BlockSpec(memory_space=pltpu.VMEM),
            pl.BlockSpec(memory_space=pltpu.VMEM),
            pl.BlockSpec(memory_space=pltpu.SMEM),
            pl.BlockSpec(memory_space=pltpu.SMEM),
        ],
        out_specs=pl.BlockSpec(memory_space=pltpu.VMEM),
        scratch_shapes=[
            pltpu.VMEM((m_total, k_per), jnp.float8_e5m2),
            pltpu.VMEM((m_per, k_total), jnp.float8_e5m2),
            pltpu.SemaphoreType.DMA((N_DEV,)),
            pltpu.SemaphoreType.DMA((N_DEV,)),
        ],
        compiler_params=pltpu.CompilerParams(
            collective_id=0, vmem_limit_bytes=100 * 1024 * 1024
        ),
    )(x, w_mat, scale_x, scale_w)


# device time: 36079 ns/iter; 1.2376x vs baseline; 1.2376x over previous
import functools

import jax
import jax.numpy as jnp
from jax import lax
from jax.experimental import pallas as pl
from jax.experimental.pallas import tpu as pltpu

N_DEV = 16


def kernel(x, w_mat, scale_x, scale_w):
    m_total, k_per = x.shape
    k_total, n = w_mat.shape
    m_per = m_total // N_DEV

    def body(x_ref, w_hbm, sx_ref, sw_ref, out_ref,
             x8_ref, xfull_ref, wtile_ref, send_sems, recv_sems, w_sems):
        my = lax.axis_index("i")

        def src_of(d):
            return lax.rem(my - d + N_DEV, N_DEV)

        def w_copy(d, slot):
            e = src_of(d)
            return pltpu.make_async_copy(
                w_hbm.at[pl.ds(e * m_per, m_per), :],
                wtile_ref.at[slot],
                w_sems.at[slot],
            )

        w_copy(0, 0).start()

        x8_ref[...] = x_ref[...].astype(jnp.float8_e5m2)

        barrier = pltpu.get_barrier_semaphore()
        for d in range(1, N_DEV):
            peer = lax.rem(my + d, N_DEV)
            pl.semaphore_signal(barrier, inc=1, device_id=(peer,),
                                device_id_type=pl.DeviceIdType.MESH)
        pl.semaphore_wait(barrier, N_DEV - 1)

        sends = []
        for d in range(1, N_DEV):
            peer = lax.rem(my + d, N_DEV)
            rdma = pltpu.make_async_remote_copy(
                src_ref=x8_ref.at[pl.ds(peer * m_per, m_per), :],
                dst_ref=xfull_ref.at[:, pl.ds(my * k_per, k_per)],
                send_sem=send_sems.at[d],
                recv_sem=recv_sems.at[my],
                device_id=(peer,),
                device_id_type=pl.DeviceIdType.MESH,
            )
            rdma.start()
            sends.append(rdma)

        xfull_ref[:, pl.ds(my * k_per, k_per)] = x8_ref[pl.ds(my * m_per, m_per), :]

        for d in range(N_DEV):
            slot = d % 2
            if d + 1 < N_DEV:
                w_copy(d + 1, (d + 1) % 2).start()
            w_copy(d, slot).wait()
            e = src_of(d)
            if d > 0:
                recv = pltpu.make_async_remote_copy(
                    src_ref=x8_ref.at[pl.ds(0, m_per), :],
                    dst_ref=xfull_ref.at[:, pl.ds(e * k_per, k_per)],
                    send_sem=send_sems.at[d],
                    recv_sem=recv_sems.at[e],
                    device_id=(e,),
                    device_id_type=pl.DeviceIdType.MESH,
                )
                recv.wait_recv()
            xblk = xfull_ref[:, pl.ds(e * k_per, k_per)]
            wblk = wtile_ref[slot].astype(jnp.float8_e5m2)
            partial = lax.dot_general(
                xblk, wblk,
                dimension_numbers=(((1,), (0,)), ((), ())),
                preferred_element_type=jnp.float32,
            )
            if d == 0:
                out_ref[...] = partial
            else:
                out_ref[...] = out_ref[...] + partial

        for rdma in sends:
            rdma.wait_send()

        scale = sx_ref[0] * sw_ref[0]
        out_ref[...] = jnp.maximum(out_ref[...] * scale, 0.0)

        @functools.partial(pl.run_scoped, sem=pltpu.SemaphoreType.REGULAR)
        def _(sem):
            for d in range(1, N_DEV):
                peer = lax.rem(my + d, N_DEV)
                pl.semaphore_signal(sem, inc=1, device_id=(peer,),
                                    device_id_type=pl.DeviceIdType.MESH)
            pl.semaphore_wait(sem, N_DEV - 1)

    return pl.pallas_call(
        body,
        out_shape=jax.ShapeDtypeStruct((m_per, n), jnp.float32),
        in_specs=[
            pl.BlockSpec(memory_space=pltpu.VMEM),
            pl.BlockSpec(memory_space=pltpu.MemorySpace.HBM),
            pl.BlockSpec(memory_space=pltpu.SMEM),
            pl.BlockSpec(memory_space=pltpu.SMEM),
        ],
        out_specs=pl.BlockSpec(memory_space=pltpu.VMEM),
        scratch_shapes=[
            pltpu.VMEM((m_total, k_per), jnp.float8_e5m2),
            pltpu.VMEM((m_per, k_total), jnp.float8_e5m2),
            pltpu.VMEM((2, m_per, n), jnp.float32),
            pltpu.SemaphoreType.DMA((N_DEV,)),
            pltpu.SemaphoreType.DMA((N_DEV,)),
            pltpu.SemaphoreType.DMA((2,)),
        ],
        compiler_params=pltpu.CompilerParams(
            collective_id=0, vmem_limit_bytes=100 * 1024 * 1024
        ),
    )(x, w_mat, scale_x, scale_w)


# device time: 31134 ns/iter; 1.4342x vs baseline; 1.1588x over previous
import jax
import jax.numpy as jnp
from jax import lax
from jax.experimental import pallas as pl
from jax.experimental.pallas import tpu as pltpu

N_DEV = 16
N_GRP = 4
GRP = N_DEV // N_GRP


def kernel(x, w_mat, scale_x, scale_w):
    m_total, k_per = x.shape
    k_total, n = w_mat.shape
    m_per = m_total // N_DEV

    def body(x_ref, w_hbm, sx_ref, sw_ref, out_ref,
             x8_ref, xfull_ref, wtile_ref, send_sems, recv_sems, w_sems,
             exit_sem):
        my = lax.axis_index("i")

        def w_copy(p):
            e = lax.rem(my - p + N_DEV, N_DEV)
            return pltpu.make_async_copy(
                w_hbm.at[pl.ds(e * m_per, m_per), :],
                wtile_ref.at[p // GRP, pl.ds((p % GRP) * m_per, m_per), :],
                w_sems.at[p // GRP, p % GRP],
            )

        for p in range(N_DEV):
            w_copy(p).start()

        barrier = pltpu.get_barrier_semaphore()
        for d in range(1, N_DEV):
            peer = lax.rem(my + d, N_DEV)
            pl.semaphore_signal(barrier, inc=1, device_id=(peer,),
                                device_id_type=pl.DeviceIdType.MESH)

        x8_ref[...] = x_ref[...].astype(jnp.float8_e5m2)
        xfull_ref[:, pl.ds(0, k_per)] = x8_ref[pl.ds(my * m_per, m_per), :]

        pl.semaphore_wait(barrier, N_DEV - 1)

        sends = []
        for d in range(1, N_DEV):
            peer = lax.rem(my + d, N_DEV)
            rdma = pltpu.make_async_remote_copy(
                src_ref=x8_ref.at[pl.ds(peer * m_per, m_per), :],
                dst_ref=xfull_ref.at[:, pl.ds(d * k_per, k_per)],
                send_sem=send_sems.at[d],
                recv_sem=recv_sems.at[d],
                device_id=(peer,),
                device_id_type=pl.DeviceIdType.MESH,
            )
            rdma.start()
            sends.append(rdma)

        for g in range(N_GRP):
            for p in range(g * GRP, (g + 1) * GRP):
                w_copy(p).wait()
            for p in range(g * GRP, (g + 1) * GRP):
                if p == 0:
                    continue
                recv = pltpu.make_async_remote_copy(
                    src_ref=x8_ref.at[pl.ds(0, m_per), :],
                    dst_ref=xfull_ref.at[:, pl.ds(p * k_per, k_per)],
                    send_sem=send_sems.at[p],
                    recv_sem=recv_sems.at[p],
                    device_id=(p,),
                    device_id_type=pl.DeviceIdType.MESH,
                )
                recv.wait_recv()
            xblk = xfull_ref[:, pl.ds(g * GRP * k_per, GRP * k_per)]
            wblk = wtile_ref[g].astype(jnp.float8_e5m2)
            partial = lax.dot_general(
                xblk, wblk,
                dimension_numbers=(((1,), (0,)), ((), ())),
                preferred_element_type=jnp.float32,
            )
            if g == 0:
                out_ref[...] = partial
            else:
                out_ref[...] = out_ref[...] + partial
            if g == N_GRP - 1:
                for dd in range(1, N_DEV):
                    peer2 = lax.rem(my + dd, N_DEV)
                    pl.semaphore_signal(exit_sem, inc=1, device_id=(peer2,),
                                        device_id_type=pl.DeviceIdType.MESH)

        for rdma in sends:
            rdma.wait_send()

        scale = sx_ref[0] * sw_ref[0]
        out_ref[...] = jnp.maximum(out_ref[...] * scale, 0.0)

        pl.semaphore_wait(exit_sem, N_DEV - 1)

    return pl.pallas_call(
        body,
        out_shape=jax.ShapeDtypeStruct((m_per, n), jnp.float32),
        in_specs=[
            pl.BlockSpec(memory_space=pltpu.VMEM),
            pl.BlockSpec(memory_space=pltpu.MemorySpace.HBM),
            pl.BlockSpec(memory_space=pltpu.SMEM),
            pl.BlockSpec(memory_space=pltpu.SMEM),
        ],
        out_specs=pl.BlockSpec(memory_space=pltpu.VMEM),
        scratch_shapes=[
            pltpu.VMEM((m_total, k_per), jnp.float8_e5m2),
            pltpu.VMEM((m_per, k_total), jnp.float8_e5m2),
            pltpu.VMEM((N_GRP, GRP * m_per, n), jnp.float32),
            pltpu.SemaphoreType.DMA((N_DEV,)),
            pltpu.SemaphoreType.DMA((N_DEV,)),
            pltpu.SemaphoreType.DMA((N_GRP, GRP)),
            pltpu.SemaphoreType.REGULAR,
        ],
        compiler_params=pltpu.CompilerParams(
            collective_id=0, vmem_limit_bytes=100 * 1024 * 1024
        ),
    )(x, w_mat, scale_x, scale_w)


# device time: 29799 ns/iter; 1.4984x vs baseline; 1.0448x over previous
import jax
import jax.numpy as jnp
from jax import lax
from jax.experimental import pallas as pl
from jax.experimental.pallas import tpu as pltpu

N_DEV = 16
N_GRP = 4
GRP = N_DEV // N_GRP


def kernel(x, w_mat, scale_x, scale_w):
    m_total, k_per = x.shape
    k_total, n = w_mat.shape
    m_per = m_total // N_DEV

    def body(x_ref, w_hbm, sx_ref, sw_ref, out_hbm,
             acc_ref, x8_ref, xfull_ref, wtile_ref, send_sems, recv_sems,
             w_sems, out_sem, exit_sem):
        my = lax.axis_index("i")

        def w_copy(p):
            e = lax.rem(my - p + N_DEV, N_DEV)
            return pltpu.make_async_copy(
                w_hbm.at[pl.ds(e * m_per, m_per), :],
                wtile_ref.at[p // GRP, pl.ds((p % GRP) * m_per, m_per), :],
                w_sems.at[p // GRP, p % GRP],
            )

        for p in range(N_DEV):
            w_copy(p).start()

        barrier = pltpu.get_barrier_semaphore()
        for d in range(1, N_DEV):
            peer = lax.rem(my + d, N_DEV)
            pl.semaphore_signal(barrier, inc=1, device_id=(peer,),
                                device_id_type=pl.DeviceIdType.MESH)

        x8_ref[...] = x_ref[...].astype(jnp.float8_e5m2)
        xfull_ref[:, pl.ds(0, k_per)] = x8_ref[pl.ds(my * m_per, m_per), :]

        pl.semaphore_wait(barrier, N_DEV - 1)

        sends = []
        for d in range(1, N_DEV):
            peer = lax.rem(my + d, N_DEV)
            rdma = pltpu.make_async_remote_copy(
                src_ref=x8_ref.at[pl.ds(peer * m_per, m_per), :],
                dst_ref=xfull_ref.at[:, pl.ds(d * k_per, k_per)],
                send_sem=send_sems.at[d],
                recv_sem=recv_sems.at[d],
                device_id=(peer,),
                device_id_type=pl.DeviceIdType.MESH,
            )
            rdma.start()
            sends.append(rdma)

        for g in range(N_GRP):
            for p in range(g * GRP, (g + 1) * GRP):
                w_copy(p).wait()
            for p in range(g * GRP, (g + 1) * GRP):
                if p == 0:
                    continue
                recv = pltpu.make_async_remote_copy(
                    src_ref=x8_ref.at[pl.ds(0, m_per), :],
                    dst_ref=xfull_ref.at[:, pl.ds(p * k_per, k_per)],
                    send_sem=send_sems.at[p],
                    recv_sem=recv_sems.at[p],
                    device_id=(p,),
                    device_id_type=pl.DeviceIdType.MESH,
                )
                recv.wait_recv()
            xblk = xfull_ref[:, pl.ds(g * GRP * k_per, GRP * k_per)]
            wblk = wtile_ref[g].astype(jnp.float8_e5m2)
            partial = lax.dot_general(
                xblk, wblk,
                dimension_numbers=(((1,), (0,)), ((), ())),
                preferred_element_type=jnp.float32,
            )
            if g == 0:
                acc_ref[...] = partial
            else:
                acc_ref[...] = acc_ref[...] + partial
            if g == N_GRP - 1:
                for dd in range(1, N_DEV):
                    peer2 = lax.rem(my + dd, N_DEV)
                    pl.semaphore_signal(exit_sem, inc=1, device_id=(peer2,),
                                        device_id_type=pl.DeviceIdType.MESH)

        for rdma in sends:
            rdma.wait_send()

        scale = sx_ref[0] * sw_ref[0]
        acc_ref[...] = jnp.maximum(acc_ref[...] * scale, 0.0)
        out_dma = pltpu.make_async_copy(acc_ref, out_hbm, out_sem)
        out_dma.start()
        out_dma.wait()

        pl.semaphore_wait(exit_sem, N_DEV - 1)

    return pl.pallas_call(
        body,
        out_shape=jax.ShapeDtypeStruct((m_per, n), jnp.float32),
        in_specs=[
            pl.BlockSpec(memory_space=pltpu.VMEM),
            pl.BlockSpec(memory_space=pltpu.MemorySpace.HBM),
            pl.BlockSpec(memory_space=pltpu.SMEM),
            pl.BlockSpec(memory_space=pltpu.SMEM),
        ],
        out_specs=pl.BlockSpec(memory_space=pltpu.MemorySpace.HBM),
        scratch_shapes=[
            pltpu.VMEM((m_per, n), jnp.float32),
            pltpu.VMEM((m_total, k_per), jnp.float8_e5m2),
            pltpu.VMEM((m_per, k_total), jnp.float8_e5m2),
            pltpu.VMEM((N_GRP, GRP * m_per, n), jnp.float32),
            pltpu.SemaphoreType.DMA((N_DEV,)),
            pltpu.SemaphoreType.DMA((N_DEV,)),
            pltpu.SemaphoreType.DMA((N_GRP, GRP)),
            pltpu.SemaphoreType.DMA,
            pltpu.SemaphoreType.REGULAR,
        ],
        compiler_params=pltpu.CompilerParams(
            collective_id=0, vmem_limit_bytes=100 * 1024 * 1024
        ),
    )(x, w_mat, scale_x, scale_w)
